# baseline (speedup 1.0000x reference)
import functools

import jax
import jax.numpy as jnp
from jax import lax
from jax.experimental import pallas as pl
from jax.experimental.pallas import tpu as pltpu
from jax.experimental.pallas import tpu_sc as plsc

VOCAB = 1000000
D = 64
DP = 128
BATCH = 4096
HIST = 200

NC = 2
NS = 16
NW = NC * NS

B = BATCH * HIST
B_PER_W = B // NW
CH = 128
N_CH = B_PER_W // CH
NBUF = 4


def _make_kernel():
  mesh = plsc.VectorSubcoreMesh(core_axis_name="c", subcore_axis_name="s")

  @functools.partial(
      pl.kernel,
      mesh=mesh,
      out_type=jax.ShapeDtypeStruct((B, D), jnp.float32),
      scratch_types=[
          pltpu.VMEM((N_CH, CH), jnp.int32),
          pltpu.VMEM((NBUF, CH, DP), jnp.float32),
      ] + [pltpu.SemaphoreType.DMA] * (2 * NBUF),
      compiler_params=pltpu.CompilerParams(use_tc_tiling_on_sc=False),
  )
  def k(idx_hbm, table_hbm, out_hbm, idx_v, rows_v, *sems):
    gsems, wsems = sems[:NBUF], sems[NBUF:]
    cid = lax.axis_index("c")
    sid = lax.axis_index("s")
    wid = sid * NC + cid
    base = wid * B_PER_W

    pltpu.sync_copy(idx_hbm.at[wid], idx_v)

    def start_gather(chunk, b):
      pltpu.make_async_copy(
          table_hbm.at[idx_v.at[chunk]], rows_v.at[b], gsems[b]
      ).start()

    def wait_gather(b):
      pltpu.make_async_copy(
          table_hbm.at[idx_v.at[0]], rows_v.at[b], gsems[b]
      ).wait()

    def start_write(j, b):
      pltpu.make_async_copy(
          rows_v.at[b, :, pl.ds(0, D)],
          out_hbm.at[pl.ds(base + j * CH, CH)], wsems[b]
      ).start()

    def wait_write(b):
      pltpu.make_async_copy(
          rows_v.at[b, :, pl.ds(0, D)],
          out_hbm.at[pl.ds(base, CH)], wsems[b]
      ).wait()

    for b in range(NBUF):
      start_gather(b, b)

    def lap(t, carry):
      j0 = t * NBUF
      for b in range(NBUF):
        j = j0 + b
        wait_gather(b)
        start_write(j, b)
        nxt = j + NBUF

        @pl.when(nxt < N_CH)
        def _():
          wait_write(b)
          start_gather(nxt, b)

      return carry

    lax.fori_loop(0, N_CH // NBUF, lap, 0, unroll=False)

    for b in range(NBUF):
      wait_write(b)

  return k


_gather_kernel = _make_kernel()


@jax.jit
def kernel(indices, table):
  table_p = jnp.pad(table, ((0, 0), (0, DP - D)))
  idx = indices.reshape(NW, N_CH, CH)
  out = _gather_kernel(idx, table_p)
  return out.reshape(BATCH, HIST, D)


# speedup vs baseline: 1.2306x; 1.2306x over previous
import functools

import jax
import jax.numpy as jnp
from jax import lax
from jax.experimental import pallas as pl
from jax.experimental.pallas import tpu as pltpu
from jax.experimental.pallas import tpu_sc as plsc

VOCAB = 1000000
D = 64
DP = 128
BATCH = 4096
HIST = 200

NC = 2
NS = 16
NW = NC * NS

B = BATCH * HIST
B_PER_W = B // NW
CH = 128
N_CH = B_PER_W // CH
NBUF = 4


def _make_kernel():
  mesh = plsc.VectorSubcoreMesh(core_axis_name="c", subcore_axis_name="s")

  @functools.partial(
      pl.kernel,
      mesh=mesh,
      out_type=jax.ShapeDtypeStruct((B, DP), jnp.float32),
      scratch_types=[
          pltpu.VMEM((N_CH, CH), jnp.int32),
          pltpu.VMEM((NBUF, CH, DP), jnp.float32),
      ] + [pltpu.SemaphoreType.DMA] * (2 * NBUF),
      compiler_params=pltpu.CompilerParams(use_tc_tiling_on_sc=False),
  )
  def k(idx_hbm, table_hbm, out_hbm, idx_v, rows_v, *sems):
    gsems, wsems = sems[:NBUF], sems[NBUF:]
    cid = lax.axis_index("c")
    sid = lax.axis_index("s")
    wid = sid * NC + cid
    base = wid * B_PER_W

    pltpu.sync_copy(idx_hbm.at[wid], idx_v)

    def start_gather(chunk, b):
      pltpu.make_async_copy(
          table_hbm.at[idx_v.at[chunk]], rows_v.at[b], gsems[b]
      ).start()

    def wait_gather(b):
      pltpu.make_async_copy(
          table_hbm.at[idx_v.at[0]], rows_v.at[b], gsems[b]
      ).wait()

    def start_write(j, b):
      pltpu.make_async_copy(
          rows_v.at[b], out_hbm.at[pl.ds(base + j * CH, CH)], wsems[b]
      ).start()

    def wait_write(b):
      pltpu.make_async_copy(
          rows_v.at[b], out_hbm.at[pl.ds(base, CH)], wsems[b]
      ).wait()

    for b in range(NBUF):
      start_gather(b, b)

    def lap(t, carry):
      j0 = t * NBUF
      for b in range(NBUF):
        j = j0 + b
        wait_gather(b)
        start_write(j, b)
        nxt = j + NBUF

        @pl.when(nxt < N_CH)
        def _():
          wait_write(b)
          start_gather(nxt, b)

      return carry

    lax.fori_loop(0, N_CH // NBUF, lap, 0, unroll=False)

    for b in range(NBUF):
      wait_write(b)

  return k


_gather_kernel = _make_kernel()


@jax.jit
def kernel(indices, table):
  table_p = jnp.pad(table, ((0, 0), (0, DP - D)))
  idx = indices.reshape(NW, N_CH, CH)
  out = _gather_kernel(idx, table_p)
  return out.reshape(BATCH, HIST, DP)[:, :, :D]
